# baseline (speedup 1.0000x reference)
import jax
import jax.numpy as jnp
from jax.experimental import pallas as pl

_RBLK = 32


def _evolve_block(hist_ref, d_ref, z_ref, out_ref):
    z = z_ref[...]
    d = d_ref[...]
    R, S = z.shape
    T = d.shape[1]
    L = 2 * S
    hx = hist_ref[0]
    hy = hist_ref[1]
    hz = hist_ref[2]
    d2 = jnp.concatenate([d, d], axis=-1)
    hx2 = jnp.concatenate([hx, hx], axis=-1)
    hy2 = jnp.concatenate([hy, hy], axis=-1)
    hz2 = jnp.concatenate([hz, hz], axis=-1)
    zs = jnp.concatenate([z, -z], axis=-1)
    sgn = jnp.concatenate([jnp.full((R, S), -1.0, jnp.float32),
                           jnp.full((R, S), 1.0, jnp.float32)], axis=-1)
    best = jnp.full((R, L), 10.0, jnp.float32)
    px = jnp.broadcast_to(hx2[:, 0:1], (R, L))
    py = jnp.broadcast_to(hy2[:, 0:1], (R, L))
    pz = jnp.broadcast_to(hz2[:, 0:1], (R, L))
    for t in range(T):
        dcol = jnp.broadcast_to(d2[:, t:t + 1], (R, L))
        u = dcol * sgn + zs
        cond = (u >= 0.0) & (u < best)
        best = jnp.where(cond, u, best)
        px = jnp.where(cond, jnp.broadcast_to(hx2[:, t:t + 1], (R, L)), px)
        py = jnp.where(cond, jnp.broadcast_to(hy2[:, t:t + 1], (R, L)), py)
        pz = jnp.where(cond, jnp.broadcast_to(hz2[:, t:t + 1], (R, L)), pz)
    vals = best[:, :S]
    mx = (px[:, S:] - px[:, :S]) / z
    my = (py[:, S:] - py[:, :S]) / z
    mz = (pz[:, S:] - pz[:, :S]) / z
    norm = jnp.sqrt(mx * mx + my * my + mz * mz)
    out_ref[0] = px[:, :S] + vals * (mx / norm)
    out_ref[1] = py[:, :S] + vals * (my / norm)
    out_ref[2] = pz[:, :S] + vals * (mz / norm)


def kernel(r_hist, distances, z_vals):
    B, T = distances.shape
    S = z_vals.shape[1]
    hist_t = jnp.transpose(r_hist, (2, 0, 1))
    z = z_vals[..., 0]
    out_t = pl.pallas_call(
        _evolve_block,
        grid=(B // _RBLK,),
        in_specs=[
            pl.BlockSpec((3, _RBLK, T), lambda i: (0, i, 0)),
            pl.BlockSpec((_RBLK, T), lambda i: (i, 0)),
            pl.BlockSpec((_RBLK, S), lambda i: (i, 0)),
        ],
        out_specs=pl.BlockSpec((3, _RBLK, S), lambda i: (0, i, 0)),
        out_shape=jax.ShapeDtypeStruct((3, B, S), jnp.float32),
    )(hist_t, distances, z)
    return jnp.transpose(out_t, (1, 2, 0))


# speedup vs baseline: 1.1366x; 1.1366x over previous
import jax
import jax.numpy as jnp
from jax.experimental import pallas as pl

_RBLK = 128


def _evolve_block(hist_ref, d_ref, z_ref, out_ref):
    z = z_ref[...]
    d = d_ref[...]
    R, S = z.shape
    T = d.shape[1]
    L = 2 * S
    hx = hist_ref[0]
    hy = hist_ref[1]
    hz = hist_ref[2]
    d2 = jnp.concatenate([d, d], axis=-1)
    hx2 = jnp.concatenate([hx, hx], axis=-1)
    hy2 = jnp.concatenate([hy, hy], axis=-1)
    hz2 = jnp.concatenate([hz, hz], axis=-1)
    zs = jnp.concatenate([z, -z], axis=-1)
    sgn = jnp.concatenate([jnp.full((R, S), -1.0, jnp.float32),
                           jnp.full((R, S), 1.0, jnp.float32)], axis=-1)
    best = jnp.full((R, L), 10.0, jnp.float32)
    px = jnp.broadcast_to(hx2[:, 0:1], (R, L))
    py = jnp.broadcast_to(hy2[:, 0:1], (R, L))
    pz = jnp.broadcast_to(hz2[:, 0:1], (R, L))
    for t in range(T):
        dcol = jnp.broadcast_to(d2[:, t:t + 1], (R, L))
        u = dcol * sgn + zs
        cond = (u >= 0.0) & (u < best)
        best = jnp.where(cond, u, best)
        px = jnp.where(cond, jnp.broadcast_to(hx2[:, t:t + 1], (R, L)), px)
        py = jnp.where(cond, jnp.broadcast_to(hy2[:, t:t + 1], (R, L)), py)
        pz = jnp.where(cond, jnp.broadcast_to(hz2[:, t:t + 1], (R, L)), pz)
    vals = best[:, :S]
    mx = (px[:, S:] - px[:, :S]) / z
    my = (py[:, S:] - py[:, :S]) / z
    mz = (pz[:, S:] - pz[:, :S]) / z
    norm = jnp.sqrt(mx * mx + my * my + mz * mz)
    out_ref[0] = px[:, :S] + vals * (mx / norm)
    out_ref[1] = py[:, :S] + vals * (my / norm)
    out_ref[2] = pz[:, :S] + vals * (mz / norm)


def kernel(r_hist, distances, z_vals):
    B, T = distances.shape
    S = z_vals.shape[1]
    hist_t = jnp.transpose(r_hist, (2, 0, 1))
    z = z_vals[..., 0]
    out_t = pl.pallas_call(
        _evolve_block,
        grid=(B // _RBLK,),
        in_specs=[
            pl.BlockSpec((3, _RBLK, T), lambda i: (0, i, 0)),
            pl.BlockSpec((_RBLK, T), lambda i: (i, 0)),
            pl.BlockSpec((_RBLK, S), lambda i: (i, 0)),
        ],
        out_specs=pl.BlockSpec((3, _RBLK, S), lambda i: (0, i, 0)),
        out_shape=jax.ShapeDtypeStruct((3, B, S), jnp.float32),
    )(hist_t, distances, z)
    return jnp.transpose(out_t, (1, 2, 0))


# speedup vs baseline: 1.1488x; 1.0107x over previous
import jax
import jax.numpy as jnp
from jax.experimental import pallas as pl

_RBLK = 128


def _evolve_block(hist_ref, d_ref, z_ref, out_ref):
    z = z_ref[...]
    d = d_ref[...]
    R, S = z.shape
    T = d.shape[1]
    L = 2 * S
    hx = hist_ref[0]
    hy = hist_ref[1]
    hz = hist_ref[2]
    zs = jnp.concatenate([z, -z], axis=-1)
    sgn = jnp.concatenate([jnp.full((R, S), -1.0, jnp.float32),
                           jnp.full((R, S), 1.0, jnp.float32)], axis=-1)
    best = jnp.full((R, L), 10.0, jnp.float32)
    px = jnp.broadcast_to(hx[:, 0:1], (R, L))
    py = jnp.broadcast_to(hy[:, 0:1], (R, L))
    pz = jnp.broadcast_to(hz[:, 0:1], (R, L))
    for t in range(T):
        dcol = jnp.broadcast_to(d[:, t:t + 1], (R, L))
        u = dcol * sgn + zs
        cond = (u >= 0.0) & (u < best)
        best = jnp.where(cond, u, best)
        px = jnp.where(cond, jnp.broadcast_to(hx[:, t:t + 1], (R, L)), px)
        py = jnp.where(cond, jnp.broadcast_to(hy[:, t:t + 1], (R, L)), py)
        pz = jnp.where(cond, jnp.broadcast_to(hz[:, t:t + 1], (R, L)), pz)
    vals = best[:, :S]
    mx = (px[:, S:] - px[:, :S]) / z
    my = (py[:, S:] - py[:, :S]) / z
    mz = (pz[:, S:] - pz[:, :S]) / z
    norm = jnp.sqrt(mx * mx + my * my + mz * mz)
    out_ref[0] = px[:, :S] + vals * (mx / norm)
    out_ref[1] = py[:, :S] + vals * (my / norm)
    out_ref[2] = pz[:, :S] + vals * (mz / norm)


def kernel(r_hist, distances, z_vals):
    B, T = distances.shape
    S = z_vals.shape[1]
    hist_t = jnp.transpose(r_hist, (2, 0, 1))
    z = z_vals[..., 0]
    out_t = pl.pallas_call(
        _evolve_block,
        grid=(B // _RBLK,),
        in_specs=[
            pl.BlockSpec((3, _RBLK, T), lambda i: (0, i, 0)),
            pl.BlockSpec((_RBLK, T), lambda i: (i, 0)),
            pl.BlockSpec((_RBLK, S), lambda i: (i, 0)),
        ],
        out_specs=pl.BlockSpec((3, _RBLK, S), lambda i: (0, i, 0)),
        out_shape=jax.ShapeDtypeStruct((3, B, S), jnp.float32),
    )(hist_t, distances, z)
    return jnp.transpose(out_t, (1, 2, 0))
